# baseline (speedup 1.0000x reference)
import functools

import jax
import jax.numpy as jnp
from jax import lax
from jax.experimental import pallas as pl
from jax.experimental.pallas import tpu as pltpu
from jax.experimental.pallas import tpu_sc as plsc

_B = 4096
_S = 256
_V = 1000
_VP = 1024
_E = 128

_H = _B // 2
_NW = 32
_ROWS_PER_W = _H // _NW
_CHUNK = 32
_NCHUNK = _ROWS_PER_W // _CHUNK


def _matmul_body(x_ref, w_ref, b_ref, o_ref):
    o_ref[...] = lax.dot_general(
        x_ref[...], w_ref[...], (((1,), (1,)), ((), ())),
        preferred_element_type=jnp.float32) + b_ref[...]


def _half_logits(x, w, b, phase):
    bt = 512
    blk0 = phase * (_H // bt)
    return pl.pallas_call(
        _matmul_body,
        grid=(_H // bt,),
        in_specs=[
            pl.BlockSpec((bt, _E), lambda i: (i + blk0, 0)),
            pl.BlockSpec((_VP, _E), lambda i: (0, 0)),
            pl.BlockSpec((1, _VP), lambda i: (0, 0)),
        ],
        out_specs=pl.BlockSpec((bt, _VP), lambda i: (i, 0)),
        out_shape=jax.ShapeDtypeStruct((_H, _VP), jnp.float32),
    )(x, w, b)


def _make_sc_gather(phase):
    def _sc_gather_body(logits_hbm, ids_hbm, out_hbm,
                        lv0, lv1, iv0, iv1, ov, sl0, sl1, si0, si1):
        lvs, ivs, sls, sis = (lv0, lv1), (iv0, iv1), (sl0, sl1), (si0, si1)
        wid = lax.axis_index("s") * 2 + lax.axis_index("c")
        base = wid * _ROWS_PER_W

        def start(c):
            r0 = base + c * _CHUNK
            return (
                pltpu.async_copy(logits_hbm.at[pl.ds(r0, _CHUNK), :],
                                 lvs[c % 2], sls[c % 2]),
                pltpu.async_copy(
                    ids_hbm.at[pl.ds(phase * _H + r0, _CHUNK), :],
                    ivs[c % 2], sis[c % 2]),
            )

        pending = start(0)
        for c in range(_NCHUNK):
            for cp in pending:
                cp.wait()
            if c + 1 < _NCHUNK:
                pending = start(c + 1)
            lv, iv = lvs[c % 2], ivs[c % 2]

            @plsc.parallel_loop(0, _CHUNK)
            def row_body(r):
                rvec = jnp.broadcast_to(r, (16,)).astype(jnp.int32)
                for j in range(_S // 16):
                    col = iv[r, pl.ds(j * 16, 16)]
                    ov[r, pl.ds(j * 16, 16)] = plsc.load_gather(
                        lv, [rvec, col])

            pltpu.sync_copy(ov,
                            out_hbm.at[pl.ds(base + c * _CHUNK, _CHUNK), :])

    return functools.partial(
        pl.kernel,
        mesh=plsc.VectorSubcoreMesh(core_axis_name="c", subcore_axis_name="s"),
        out_type=jax.ShapeDtypeStruct((_H, _S), jnp.float32),
        scratch_types=[
            pltpu.VMEM((_CHUNK, _VP), jnp.float32),
            pltpu.VMEM((_CHUNK, _VP), jnp.float32),
            pltpu.VMEM((_CHUNK, _S), jnp.int32),
            pltpu.VMEM((_CHUNK, _S), jnp.int32),
            pltpu.VMEM((_CHUNK, _S), jnp.float32),
            pltpu.SemaphoreType.DMA,
            pltpu.SemaphoreType.DMA,
            pltpu.SemaphoreType.DMA,
            pltpu.SemaphoreType.DMA,
        ],
        compiler_params=pltpu.CompilerParams(needs_layout_passes=False),
    )(_sc_gather_body)


_sc_gather_0 = _make_sc_gather(0)
_sc_gather_1 = _make_sc_gather(1)


def _loss_body(g_ref, o_ref):
    x = g_ref[...]
    m = jnp.max(x, axis=1, keepdims=True)
    lse = jnp.log(jnp.sum(jnp.exp(x - m), axis=1, keepdims=True)) + m
    loss = jnp.sum(lse - x[:, 0:1]) * (1.0 / _H)
    o_ref[...] = jnp.full((1, 1), loss, jnp.float32)


def _loss(g):
    return pl.pallas_call(
        _loss_body,
        out_shape=jax.ShapeDtypeStruct((1, 1), jnp.float32),
    )(g)


def kernel(inputs, labels, weight, bias, sample_ids):
    b2 = bias.reshape(1, _V)
    full0 = _half_logits(inputs, weight, b2, 0)
    g0 = _sc_gather_0(full0, sample_ids)
    full1 = _half_logits(inputs, weight, b2, 1)
    g1 = _sc_gather_1(full1, sample_ids)
    return (_loss(g0)[0, 0] + _loss(g1)[0, 0]) * 0.5


# speedup vs baseline: 1.0832x; 1.0832x over previous
import functools

import jax
import jax.numpy as jnp
from jax import lax
from jax.experimental import pallas as pl
from jax.experimental.pallas import tpu as pltpu
from jax.experimental.pallas import tpu_sc as plsc

_B = 4096
_S = 256
_V = 1000
_VP = 1024
_E = 128

_NW = 32
_ROWS_PER_W = _B // _NW
_CHUNK = 32
_NCHUNK = _ROWS_PER_W // _CHUNK


def _matmul_body(x_ref, w_ref, b_ref, o_ref):
    o_ref[...] = lax.dot_general(
        x_ref[...], w_ref[...], (((1,), (1,)), ((), ())),
        preferred_element_type=jnp.float32) + b_ref[...]


def _full_logits(x, w, b):
    bt = 512
    return pl.pallas_call(
        _matmul_body,
        grid=(_B // bt,),
        in_specs=[
            pl.BlockSpec((bt, _E), lambda i: (i, 0)),
            pl.BlockSpec((_VP, _E), lambda i: (0, 0)),
            pl.BlockSpec((1, _VP), lambda i: (0, 0)),
        ],
        out_specs=pl.BlockSpec((bt, _VP), lambda i: (i, 0)),
        out_shape=jax.ShapeDtypeStruct((_B, _VP), jnp.float32),
    )(x, w, b)


def _sc_gather_body(logits_hbm, ids_hbm, out_hbm,
                    lv0, lv1, iv0, iv1, ov, sl0, sl1, si0, si1):
    lvs, ivs, sls, sis = (lv0, lv1), (iv0, iv1), (sl0, sl1), (si0, si1)
    wid = lax.axis_index("s") * 2 + lax.axis_index("c")
    base = wid * _ROWS_PER_W

    def start(c):
        r0 = base + c * _CHUNK
        return (
            pltpu.async_copy(logits_hbm.at[pl.ds(r0, _CHUNK), :],
                             lvs[c % 2], sls[c % 2]),
            pltpu.async_copy(ids_hbm.at[pl.ds(r0, _CHUNK), :],
                             ivs[c % 2], sis[c % 2]),
        )

    pending = start(0)
    for c in range(_NCHUNK):
        for cp in pending:
            cp.wait()
        if c + 1 < _NCHUNK:
            pending = start(c + 1)
        lv, iv = lvs[c % 2], ivs[c % 2]

        @plsc.parallel_loop(0, _CHUNK)
        def row_body(r):
            rvec = jnp.broadcast_to(r, (16,)).astype(jnp.int32)
            for j in range(_S // 16):
                col = iv[r, pl.ds(j * 16, 16)]
                ov[r, pl.ds(j * 16, 16)] = plsc.load_gather(lv, [rvec, col])

        pltpu.sync_copy(ov, out_hbm.at[pl.ds(base + c * _CHUNK, _CHUNK), :])


_sc_gather = functools.partial(
    pl.kernel,
    mesh=plsc.VectorSubcoreMesh(core_axis_name="c", subcore_axis_name="s"),
    out_type=jax.ShapeDtypeStruct((_B, _S), jnp.float32),
    scratch_types=[
        pltpu.VMEM((_CHUNK, _VP), jnp.float32),
        pltpu.VMEM((_CHUNK, _VP), jnp.float32),
        pltpu.VMEM((_CHUNK, _S), jnp.int32),
        pltpu.VMEM((_CHUNK, _S), jnp.int32),
        pltpu.VMEM((_CHUNK, _S), jnp.float32),
        pltpu.SemaphoreType.DMA,
        pltpu.SemaphoreType.DMA,
        pltpu.SemaphoreType.DMA,
        pltpu.SemaphoreType.DMA,
    ],
    compiler_params=pltpu.CompilerParams(needs_layout_passes=False),
)(_sc_gather_body)


def _loss_body(g_ref, o_ref):
    x = g_ref[...]
    m = jnp.max(x, axis=1, keepdims=True)
    lse = jnp.log(jnp.sum(jnp.exp(x - m), axis=1, keepdims=True)) + m
    loss = jnp.sum(lse - x[:, 0:1]) * (1.0 / _B)
    o_ref[...] = jnp.full((1, 1), loss, jnp.float32)


def _loss(g):
    return pl.pallas_call(
        _loss_body,
        out_shape=jax.ShapeDtypeStruct((1, 1), jnp.float32),
    )(g)


def kernel(inputs, labels, weight, bias, sample_ids):
    full = _full_logits(inputs, weight, bias.reshape(1, _V))
    gathered = _sc_gather(full, sample_ids)
    return _loss(gathered)[0, 0]


# speedup vs baseline: 1.1281x; 1.0415x over previous
import functools

import jax
import jax.numpy as jnp
from jax import lax
from jax.experimental import pallas as pl
from jax.experimental.pallas import tpu as pltpu
from jax.experimental.pallas import tpu_sc as plsc

_B = 4096
_S = 256
_V = 1000
_VP = 1024
_E = 128

_NW = 32
_ROWS_PER_W = _B // _NW
_CHUNK = 32
_NCHUNK = _ROWS_PER_W // _CHUNK


def _matmul_body(x_ref, w_ref, b_ref, o_ref):
    o_ref[...] = lax.dot_general(
        x_ref[...], w_ref[...], (((1,), (1,)), ((), ())),
        preferred_element_type=jnp.float32) + b_ref[...]


def _full_logits(x, w, b):
    bt = 1024
    return pl.pallas_call(
        _matmul_body,
        grid=(_B // bt,),
        in_specs=[
            pl.BlockSpec((bt, _E), lambda i: (i, 0)),
            pl.BlockSpec((_VP, _E), lambda i: (0, 0)),
            pl.BlockSpec((1, _VP), lambda i: (0, 0)),
        ],
        out_specs=pl.BlockSpec((bt, _VP), lambda i: (i, 0)),
        out_shape=jax.ShapeDtypeStruct((_B, _VP), jnp.float32),
    )(x, w, b)


def _sc_gather_body(logits_hbm, ids_hbm, out_hbm,
                    lv0, lv1, iv0, iv1, ov, sl0, sl1, si0, si1):
    lvs, ivs, sls, sis = (lv0, lv1), (iv0, iv1), (sl0, sl1), (si0, si1)
    wid = lax.axis_index("s") * 2 + lax.axis_index("c")
    base = wid * _ROWS_PER_W

    def start(c):
        r0 = base + c * _CHUNK
        return (
            pltpu.async_copy(logits_hbm.at[pl.ds(r0, _CHUNK), :],
                             lvs[c % 2], sls[c % 2]),
            pltpu.async_copy(ids_hbm.at[pl.ds(r0, _CHUNK), :],
                             ivs[c % 2], sis[c % 2]),
        )

    pending = start(0)
    for c in range(_NCHUNK):
        for cp in pending:
            cp.wait()
        if c + 1 < _NCHUNK:
            pending = start(c + 1)
        lv, iv = lvs[c % 2], ivs[c % 2]

        @plsc.parallel_loop(0, _CHUNK)
        def row_body(r):
            rvec = jnp.broadcast_to(r, (16,)).astype(jnp.int32)
            for j in range(_S // 16):
                col = iv[r, pl.ds(j * 16, 16)]
                ov[r, pl.ds(j * 16, 16)] = plsc.load_gather(lv, [rvec, col])

        pltpu.sync_copy(ov, out_hbm.at[pl.ds(base + c * _CHUNK, _CHUNK), :])


_sc_gather = functools.partial(
    pl.kernel,
    mesh=plsc.VectorSubcoreMesh(core_axis_name="c", subcore_axis_name="s"),
    out_type=jax.ShapeDtypeStruct((_B, _S), jnp.float32),
    scratch_types=[
        pltpu.VMEM((_CHUNK, _VP), jnp.float32),
        pltpu.VMEM((_CHUNK, _VP), jnp.float32),
        pltpu.VMEM((_CHUNK, _S), jnp.int32),
        pltpu.VMEM((_CHUNK, _S), jnp.int32),
        pltpu.VMEM((_CHUNK, _S), jnp.float32),
        pltpu.SemaphoreType.DMA,
        pltpu.SemaphoreType.DMA,
        pltpu.SemaphoreType.DMA,
        pltpu.SemaphoreType.DMA,
    ],
    compiler_params=pltpu.CompilerParams(needs_layout_passes=False),
)(_sc_gather_body)


def _loss_body(g_ref, o_ref):
    x = g_ref[...]
    m = jnp.max(x, axis=1, keepdims=True)
    lse = jnp.log(jnp.sum(jnp.exp(x - m), axis=1, keepdims=True)) + m
    loss = jnp.sum(lse - x[:, 0:1]) * (1.0 / _B)
    o_ref[...] = jnp.full((1, 1), loss, jnp.float32)


def _loss(g):
    return pl.pallas_call(
        _loss_body,
        out_shape=jax.ShapeDtypeStruct((1, 1), jnp.float32),
    )(g)


def kernel(inputs, labels, weight, bias, sample_ids):
    full = _full_logits(inputs, weight, bias.reshape(1, _V))
    gathered = _sc_gather(full, sample_ids)
    return _loss(gathered)[0, 0]
